# Initial kernel scaffold; baseline (speedup 1.0000x reference)
import jax
import jax.numpy as jnp
from jax.experimental import pallas as pl


def kernel(source_ufea, target_ufea, source_user_idx, source_item_idx, source_vals, target_user_idx, target_item_idx, target_vals, W_gc1, b_gc1, W_gc2, b_gc2, W_gc3m, b_gc3m, W_gc3s, b_gc3s, W_gc4m, b_gc4m, W_gc4s, b_gc4s, W_sum, b_sum, W_sls, b_sls, W_tum, b_tum, W_tls, b_tls):
    raise NotImplementedError("write your pallas kernel here")


# speedup vs baseline: 1.8530x; 1.8530x over previous
import functools
import math

import jax
import jax.numpy as jnp
from jax import lax
from jax.experimental import pallas as pl
from jax.experimental.pallas import tpu as pltpu
from jax.experimental.pallas import tpu_sc as plsc

N_USR = 29999
N_ITM = 20000
E = 480000
D = 128
ALPHA = 0.1
RATE = 0.5
TGT_USERS = 30000
TOTAL_USERS = 50000

CH = 64
E_PAD = 480256
U_PAD = 30208
I_PAD = 20224
HALF_U = U_PAD // 2
HALF_I = I_PAD // 2


def _leaky(x):
    return jnp.where(x > 0, x, ALPHA * x)


@functools.lru_cache(maxsize=None)
def _make_spmm(out_rows, half):
    ept = E_PAD // 16
    n_chunks = ept // CH
    zpt = half // 16
    mesh = plsc.VectorSubcoreMesh(core_axis_name="c", subcore_axis_name="s")

    @functools.partial(
        pl.kernel,
        out_type=jax.ShapeDtypeStruct((out_rows, D), jnp.float32),
        mesh=mesh,
        scratch_types=[
            pltpu.VMEM((CH,), jnp.int32),
            pltpu.VMEM((CH,), jnp.int32),
            pltpu.VMEM((CH,), jnp.float32),
            pltpu.VMEM((CH,), jnp.int32),
            pltpu.VMEM((CH, D), jnp.float32),
            pltpu.VMEM_SHARED((half, D), jnp.float32),
            pltpu.SemaphoreType.DMA,
        ],
    )
    def spmm(x_hbm, cols_hbm, rows_hbm, vals_hbm, zeros_hbm, out_hbm,
             cols_v, rows_v, vals_v, loc_v, gath_v, acc, sem):
        c = lax.axis_index("c")
        s = lax.axis_index("s")
        base = c * half
        pltpu.sync_copy(zeros_hbm.at[pl.ds(s * zpt, zpt)],
                        acc.at[pl.ds(s * zpt, zpt)])
        plsc.subcore_barrier()

        def chunk(i, carry):
            off = s * ept + i * CH
            pltpu.sync_copy(cols_hbm.at[pl.ds(off, CH)], cols_v)
            pltpu.sync_copy(rows_hbm.at[pl.ds(off, CH)], rows_v)
            pltpu.sync_copy(vals_hbm.at[pl.ds(off, CH)], vals_v)
            pltpu.async_copy(x_hbm.at[cols_v], gath_v, sem).wait()
            for g in range(CH // 16):
                sl = pl.ds(g * 16, 16)
                lo = rows_v[sl] - base
                inr = (lo >= 0) & (lo < half)
                loc_v[sl] = jnp.where(inr, lo, cols_v[sl] & 8191)
                vals_v[sl] = jnp.where(inr, vals_v[sl], 0.0)

            def scale(g, carry2):
                vv = vals_v[pl.ds(g * 16, 16)]
                for l in range(16):
                    v = vv[l]
                    e = g * 16 + l
                    for kk in range(D // 16):
                        sl2 = pl.ds(kk * 16, 16)
                        gath_v[e, sl2] = gath_v[e, sl2] * v
                return carry2

            lax.fori_loop(0, CH // 16, scale, 0)
            pltpu.sync_copy(gath_v, acc.at[loc_v], add=True)
            return carry

        lax.fori_loop(0, n_chunks, chunk, 0)
        plsc.subcore_barrier()
        pltpu.sync_copy(acc.at[pl.ds(s * zpt, zpt)],
                        out_hbm.at[pl.ds(base + s * zpt, zpt)])

    return spmm


def _dense1(x, W, b):
    n = x.shape[0]
    br = 2528

    def body(x_ref, w_ref, b_ref, o_ref):
        acc = jnp.dot(x_ref[...], w_ref[...],
                      preferred_element_type=jnp.float32) + b_ref[...]
        o_ref[...] = _leaky(acc)

    return pl.pallas_call(
        body,
        grid=(n // br,),
        in_specs=[pl.BlockSpec((br, D), lambda i: (i, 0)),
                  pl.BlockSpec((D, D), lambda i: (0, 0)),
                  pl.BlockSpec((1, D), lambda i: (0, 0))],
        out_specs=pl.BlockSpec((br, D), lambda i: (i, 0)),
        out_shape=jax.ShapeDtypeStruct((n, D), jnp.float32),
    )(x, W, b.reshape(1, D))


def _head(S2, uf, Wm, bm, Ws, bs, Wc1a, Wc1b, bc1, Wc2a, Wc2b, bc2):
    n = S2.shape[0]
    br = 1888

    def body(s_ref, u_ref, wm, bm_, ws, bs_, wa1, wb1, bb1, wa2, wb2, bb2,
             o1, o2):
        sv = s_ref[...]
        uv = u_ref[...]
        hm = _leaky(jnp.dot(sv, wm[...], preferred_element_type=jnp.float32)
                    + bm_[...])
        hs = _leaky(jnp.dot(sv, ws[...], preferred_element_type=jnp.float32)
                    + bs_[...])
        ub1 = jnp.dot(uv, wb1[...], preferred_element_type=jnp.float32)
        ub2 = jnp.dot(uv, wb2[...], preferred_element_type=jnp.float32)
        o1[...] = jnp.dot(hm, wa1[...], preferred_element_type=jnp.float32) \
            + ub1 + bb1[...]
        o2[...] = jnp.dot(hs, wa2[...], preferred_element_type=jnp.float32) \
            + ub2 + bb2[...]

    mat = pl.BlockSpec((D, D), lambda i: (0, 0))
    vec = pl.BlockSpec((1, D), lambda i: (0, 0))
    blk = pl.BlockSpec((br, D), lambda i: (i, 0))
    return pl.pallas_call(
        body,
        grid=(n // br,),
        in_specs=[blk, blk, mat, vec, mat, vec, mat, mat, vec, mat, mat, vec],
        out_specs=[blk, blk],
        out_shape=[jax.ShapeDtypeStruct((n, D), jnp.float32),
                   jax.ShapeDtypeStruct((n, D), jnp.float32)],
    )(S2, uf, Wm, bm.reshape(1, D), Ws, bs.reshape(1, D),
      Wc1a, Wc1b, bc1.reshape(1, D), Wc2a, Wc2b, bc2.reshape(1, D))


def _final(smp, slsp, tmp, tlsp):
    br = 2000
    nb = TOTAL_USERS // br
    a2 = 0.1 + 0.9 * math.log(2.0)
    inv2s2 = 0.5 * math.exp(-2.0 * a2)

    def body(sm_ref, tm_ref, sl_ref, tl_ref, mean_ref, ks_ref):
        i = pl.program_id(0)
        both = i < 5
        t_only = (i >= 5) & (i < 15)
        sm = sm_ref[...]
        tm = tm_ref[...]
        sl = sl_ref[...]
        tl = tl_ref[...]
        mean = jnp.where(both, RATE * sm + (1 - RATE) * tm,
                         jnp.where(t_only, (1 - RATE) * tm, RATE * sm))
        ls = jnp.where(both, RATE * sl + (1 - RATE) * tl,
                       jnp.where(t_only, (1 - RATE) * tl, RATE * sl))
        mean_ref[...] = mean
        a1 = 0.1 + 0.9 * (jnp.maximum(ls, 0.0)
                          + jnp.log(1.0 + jnp.exp(-jnp.abs(ls))))
        kl = (a2 - a1) + (jnp.exp(2.0 * a1) + mean * mean) * inv2s2 - 0.5
        part = jnp.sum(kl)

        @pl.when(i == 0)
        def _():
            ks_ref[...] = jnp.zeros_like(ks_ref)

        ks_ref[...] = ks_ref[...] + part

        @pl.when(i == nb - 1)
        def _():
            ks_ref[...] = ks_ref[...] / float(TOTAL_USERS)

    def s_map(i):
        return (jnp.where(i < 5, i, jnp.where(i >= 15, i - 10, 0)), 0)

    def t_map(i):
        return (jnp.where(i < 15, i, 0), 0)

    blk_s = pl.BlockSpec((br, D), s_map)
    blk_t = pl.BlockSpec((br, D), t_map)
    return pl.pallas_call(
        body,
        grid=(nb,),
        in_specs=[blk_s, blk_t, blk_s, blk_t],
        out_specs=[pl.BlockSpec((br, D), lambda i: (i, 0)),
                   pl.BlockSpec((1, 1), lambda i: (0, 0))],
        out_shape=[jax.ShapeDtypeStruct((TOTAL_USERS, D), jnp.float32),
                   jax.ShapeDtypeStruct((1, 1), jnp.float32)],
    )(smp, tmp, slsp, tlsp)


def _pad_rows(x, n):
    return jnp.pad(x, ((0, n - x.shape[0]), (0, 0)))


def _pad_edges(idx, n_valid):
    pad = E_PAD - E
    filler = (jnp.arange(pad, dtype=jnp.int32) % min(997, n_valid))
    return jnp.concatenate([idx.astype(jnp.int32), filler])


def _shift(x):
    return jnp.concatenate([jnp.zeros((1, D), jnp.float32), x[:TGT_USERS - 1]])


def kernel(source_ufea, target_ufea, source_user_idx, source_item_idx,
           source_vals, target_user_idx, target_item_idx, target_vals,
           W_gc1, b_gc1, W_gc2, b_gc2, W_gc3m, b_gc3m, W_gc3s, b_gc3s,
           W_gc4m, b_gc4m, W_gc4s, b_gc4s, W_sum, b_sum, W_sls, b_sls,
           W_tum, b_tum, W_tls, b_tls):
    su = _pad_rows(source_ufea, U_PAD)
    tu = _pad_rows(target_ufea, U_PAD)
    s_ui = _pad_edges(source_user_idx, N_USR)
    s_ii = _pad_edges(source_item_idx, N_ITM)
    t_ui = _pad_edges(target_user_idx, N_USR)
    t_ii = _pad_edges(target_item_idx, N_ITM)
    vpad = jnp.zeros((E_PAD - E,), jnp.float32)
    s_v = jnp.concatenate([source_vals, vpad])
    t_v = jnp.concatenate([target_vals, vpad])
    zeros_i = jnp.zeros((HALF_I, D), jnp.float32)
    zeros_u = jnp.zeros((HALF_U, D), jnp.float32)
    _spmm_item = _make_spmm(I_PAD, HALF_I)
    _spmm_user = _make_spmm(U_PAD, HALF_U)

    S1s = _spmm_item(su, s_ui, s_ii, s_v, zeros_i)
    S1t = _spmm_item(tu, t_ui, t_ii, t_v, zeros_i)
    s_ho = _dense1(S1s, W_gc1, b_gc1)
    t_ho = _dense1(S1t, W_gc2, b_gc2)
    S2s = _spmm_user(s_ho, s_ii, s_ui, s_v, zeros_u)
    S2t = _spmm_user(t_ho, t_ii, t_ui, t_v, zeros_u)

    sm, sls = _head(S2s, su, W_gc3m, b_gc3m, W_gc3s, b_gc3s,
                    W_sum[:D], W_sum[D:], b_sum, W_sls[:D], W_sls[D:], b_sls)
    tm, tls = _head(S2t, tu, W_gc4m, b_gc4m, W_gc4s, b_gc4s,
                    W_tum[:D], W_tum[D:], b_tum, W_tls[:D], W_tls[D:], b_tls)

    user, ksum = _final(_shift(sm), _shift(sls), _shift(tm), _shift(tls))
    return user, ksum.reshape(())


# speedup vs baseline: 3.0750x; 1.6595x over previous
import functools
import math

import jax
import jax.numpy as jnp
from jax import lax
from jax.experimental import pallas as pl
from jax.experimental.pallas import tpu as pltpu
from jax.experimental.pallas import tpu_sc as plsc

N_USR = 29999
N_ITM = 20000
E = 480000
D = 128
ALPHA = 0.1
RATE = 0.5
TGT_USERS = 30000
TOTAL_USERS = 50000

CH = 32
BLK_CH = 8
BLK_E = CH * BLK_CH
E_PAD = 491520
U_PAD = 30208
I_PAD = 20224
HALF_U = U_PAD // 2
HALF_I = I_PAD // 2


def _leaky(x):
    return jnp.where(x > 0, x, ALPHA * x)


@functools.lru_cache(maxsize=None)
def _make_spmm(out_rows, half):
    ept = E_PAD // 16
    n_blocks = ept // BLK_E
    zpt = half // 16
    mesh = plsc.VectorSubcoreMesh(core_axis_name="c", subcore_axis_name="s")

    @functools.partial(
        pl.kernel,
        out_type=jax.ShapeDtypeStruct((out_rows, D), jnp.float32),
        mesh=mesh,
        scratch_types=[
            pltpu.VMEM((BLK_E,), jnp.int32),
            pltpu.VMEM((BLK_E,), jnp.int32),
            pltpu.VMEM((BLK_E,), jnp.float32),
            pltpu.VMEM((CH, D), jnp.float32),
            pltpu.VMEM((CH, D), jnp.float32),
            pltpu.VMEM((CH,), jnp.int32),
            pltpu.VMEM((CH,), jnp.int32),
            pltpu.VMEM((CH,), jnp.float32),
            pltpu.VMEM((CH,), jnp.float32),
            pltpu.VMEM_SHARED((half, D), jnp.float32),
            pltpu.SemaphoreType.DMA,
            pltpu.SemaphoreType.DMA,
            pltpu.SemaphoreType.DMA,
            pltpu.SemaphoreType.DMA,
        ],
    )
    def spmm(x_hbm, cols_hbm, rows_hbm, vals_hbm, zeros_hbm, out_hbm,
             colsb, rowsb, valsb, g0, g1, l0, l1, v0, v1, acc,
             gs0, gs1, ss0, ss1):
        c = lax.axis_index("c")
        s = lax.axis_index("s")
        base = c * half
        gath = (g0, g1)
        locs = (l0, l1)
        vms = (v0, v1)
        gsem = (gs0, gs1)
        ssem = (ss0, ss1)
        pltpu.sync_copy(zeros_hbm.at[pl.ds(s * zpt, zpt)],
                        acc.at[pl.ds(s * zpt, zpt)])
        plsc.subcore_barrier()

        def block(b, carry):
            boff = s * ept + b * BLK_E
            pltpu.sync_copy(cols_hbm.at[pl.ds(boff, BLK_E)], colsb)
            pltpu.sync_copy(rows_hbm.at[pl.ds(boff, BLK_E)], rowsb)
            pltpu.sync_copy(vals_hbm.at[pl.ds(boff, BLK_E)], valsb)
            gd = {0: pltpu.async_copy(x_hbm.at[colsb.at[pl.ds(0, CH)]],
                                      gath[0], gsem[0])}
            sd = [None, None]
            for jj in range(BLK_CH):
                buf = jj & 1
                if jj < BLK_CH - 1:
                    nbuf = buf ^ 1
                    if jj >= 1:
                        sd[nbuf].wait()
                    gd[jj + 1] = pltpu.async_copy(
                        x_hbm.at[colsb.at[pl.ds((jj + 1) * CH, CH)]],
                        gath[nbuf], gsem[nbuf])
                for g in range(CH // 16):
                    slb = pl.ds(jj * CH + g * 16, 16)
                    sl = pl.ds(g * 16, 16)
                    lo = rowsb[slb] - base
                    inr = (lo >= 0) & (lo < half)
                    locs[buf][sl] = jnp.where(inr, lo, colsb[slb] & 8191)
                    vms[buf][sl] = jnp.where(inr, valsb[slb], 0.0)
                gd[jj].wait()

                def scale(g2, carry2, _vm=vms[buf], _gt=gath[buf]):
                    vv = _vm[pl.ds(g2 * 16, 16)]
                    for l in range(16):
                        v = vv[l]
                        e = g2 * 16 + l
                        for kk in range(D // 16):
                            sl2 = pl.ds(kk * 16, 16)
                            _gt[e, sl2] = _gt[e, sl2] * v
                    return carry2

                lax.fori_loop(0, CH // 16, scale, 0)
                sd[buf] = pltpu.async_copy(gath[buf], acc.at[locs[buf]],
                                           ssem[buf], add=True)
            sd[0].wait()
            sd[1].wait()
            return carry

        lax.fori_loop(0, n_blocks, block, 0)
        plsc.subcore_barrier()
        pltpu.sync_copy(acc.at[pl.ds(s * zpt, zpt)],
                        out_hbm.at[pl.ds(base + s * zpt, zpt)])

    return spmm


def _dense1(x, W, b):
    n = x.shape[0]
    br = 2528

    def body(x_ref, w_ref, b_ref, o_ref):
        acc = jnp.dot(x_ref[...], w_ref[...],
                      preferred_element_type=jnp.float32) + b_ref[...]
        o_ref[...] = _leaky(acc)

    return pl.pallas_call(
        body,
        grid=(n // br,),
        in_specs=[pl.BlockSpec((br, D), lambda i: (i, 0)),
                  pl.BlockSpec((D, D), lambda i: (0, 0)),
                  pl.BlockSpec((1, D), lambda i: (0, 0))],
        out_specs=pl.BlockSpec((br, D), lambda i: (i, 0)),
        out_shape=jax.ShapeDtypeStruct((n, D), jnp.float32),
    )(x, W, b.reshape(1, D))


def _head(S2, uf, Wm, bm, Ws, bs, Wc1a, Wc1b, bc1, Wc2a, Wc2b, bc2):
    n = S2.shape[0]
    br = 1888

    def body(s_ref, u_ref, wm, bm_, ws, bs_, wa1, wb1, bb1, wa2, wb2, bb2,
             o1, o2):
        sv = s_ref[...]
        uv = u_ref[...]
        hm = _leaky(jnp.dot(sv, wm[...], preferred_element_type=jnp.float32)
                    + bm_[...])
        hs = _leaky(jnp.dot(sv, ws[...], preferred_element_type=jnp.float32)
                    + bs_[...])
        ub1 = jnp.dot(uv, wb1[...], preferred_element_type=jnp.float32)
        ub2 = jnp.dot(uv, wb2[...], preferred_element_type=jnp.float32)
        o1[...] = jnp.dot(hm, wa1[...], preferred_element_type=jnp.float32) \
            + ub1 + bb1[...]
        o2[...] = jnp.dot(hs, wa2[...], preferred_element_type=jnp.float32) \
            + ub2 + bb2[...]

    mat = pl.BlockSpec((D, D), lambda i: (0, 0))
    vec = pl.BlockSpec((1, D), lambda i: (0, 0))
    blk = pl.BlockSpec((br, D), lambda i: (i, 0))
    return pl.pallas_call(
        body,
        grid=(n // br,),
        in_specs=[blk, blk, mat, vec, mat, vec, mat, mat, vec, mat, mat, vec],
        out_specs=[blk, blk],
        out_shape=[jax.ShapeDtypeStruct((n, D), jnp.float32),
                   jax.ShapeDtypeStruct((n, D), jnp.float32)],
    )(S2, uf, Wm, bm.reshape(1, D), Ws, bs.reshape(1, D),
      Wc1a, Wc1b, bc1.reshape(1, D), Wc2a, Wc2b, bc2.reshape(1, D))


def _final(smp, slsp, tmp, tlsp):
    br = 2000
    nb = TOTAL_USERS // br
    a2 = 0.1 + 0.9 * math.log(2.0)
    inv2s2 = 0.5 * math.exp(-2.0 * a2)

    def body(sm_ref, tm_ref, sl_ref, tl_ref, mean_ref, ks_ref):
        i = pl.program_id(0)
        both = i < 5
        t_only = (i >= 5) & (i < 15)
        sm = sm_ref[...]
        tm = tm_ref[...]
        sl = sl_ref[...]
        tl = tl_ref[...]
        mean = jnp.where(both, RATE * sm + (1 - RATE) * tm,
                         jnp.where(t_only, (1 - RATE) * tm, RATE * sm))
        ls = jnp.where(both, RATE * sl + (1 - RATE) * tl,
                       jnp.where(t_only, (1 - RATE) * tl, RATE * sl))
        mean_ref[...] = mean
        a1 = 0.1 + 0.9 * (jnp.maximum(ls, 0.0)
                          + jnp.log(1.0 + jnp.exp(-jnp.abs(ls))))
        kl = (a2 - a1) + (jnp.exp(2.0 * a1) + mean * mean) * inv2s2 - 0.5
        part = jnp.sum(kl)

        @pl.when(i == 0)
        def _():
            ks_ref[...] = jnp.zeros_like(ks_ref)

        ks_ref[...] = ks_ref[...] + part

        @pl.when(i == nb - 1)
        def _():
            ks_ref[...] = ks_ref[...] / float(TOTAL_USERS)

    def s_map(i):
        return (jnp.where(i < 5, i, jnp.where(i >= 15, i - 10, 0)), 0)

    def t_map(i):
        return (jnp.where(i < 15, i, 0), 0)

    blk_s = pl.BlockSpec((br, D), s_map)
    blk_t = pl.BlockSpec((br, D), t_map)
    return pl.pallas_call(
        body,
        grid=(nb,),
        in_specs=[blk_s, blk_t, blk_s, blk_t],
        out_specs=[pl.BlockSpec((br, D), lambda i: (i, 0)),
                   pl.BlockSpec((1, 1), lambda i: (0, 0))],
        out_shape=[jax.ShapeDtypeStruct((TOTAL_USERS, D), jnp.float32),
                   jax.ShapeDtypeStruct((1, 1), jnp.float32)],
    )(smp, tmp, slsp, tlsp)


def _pad_rows(x, n):
    return jnp.pad(x, ((0, n - x.shape[0]), (0, 0)))


def _pad_edges(idx, n_valid):
    pad = E_PAD - E
    filler = (jnp.arange(pad, dtype=jnp.int32) % min(997, n_valid))
    return jnp.concatenate([idx.astype(jnp.int32), filler])


def _shift(x):
    return jnp.concatenate([jnp.zeros((1, D), jnp.float32), x[:TGT_USERS - 1]])


def kernel(source_ufea, target_ufea, source_user_idx, source_item_idx,
           source_vals, target_user_idx, target_item_idx, target_vals,
           W_gc1, b_gc1, W_gc2, b_gc2, W_gc3m, b_gc3m, W_gc3s, b_gc3s,
           W_gc4m, b_gc4m, W_gc4s, b_gc4s, W_sum, b_sum, W_sls, b_sls,
           W_tum, b_tum, W_tls, b_tls):
    su = _pad_rows(source_ufea, U_PAD)
    tu = _pad_rows(target_ufea, U_PAD)
    s_ui = _pad_edges(source_user_idx, N_USR)
    s_ii = _pad_edges(source_item_idx, N_ITM)
    t_ui = _pad_edges(target_user_idx, N_USR)
    t_ii = _pad_edges(target_item_idx, N_ITM)
    vpad = jnp.zeros((E_PAD - E,), jnp.float32)
    s_v = jnp.concatenate([source_vals, vpad])
    t_v = jnp.concatenate([target_vals, vpad])
    zeros_i = jnp.zeros((HALF_I, D), jnp.float32)
    zeros_u = jnp.zeros((HALF_U, D), jnp.float32)
    _spmm_item = _make_spmm(I_PAD, HALF_I)
    _spmm_user = _make_spmm(U_PAD, HALF_U)

    S1s = _spmm_item(su, s_ui, s_ii, s_v, zeros_i)
    S1t = _spmm_item(tu, t_ui, t_ii, t_v, zeros_i)
    s_ho = _dense1(S1s, W_gc1, b_gc1)
    t_ho = _dense1(S1t, W_gc2, b_gc2)
    S2s = _spmm_user(s_ho, s_ii, s_ui, s_v, zeros_u)
    S2t = _spmm_user(t_ho, t_ii, t_ui, t_v, zeros_u)

    sm, sls = _head(S2s, su, W_gc3m, b_gc3m, W_gc3s, b_gc3s,
                    W_sum[:D], W_sum[D:], b_sum, W_sls[:D], W_sls[D:], b_sls)
    tm, tls = _head(S2t, tu, W_gc4m, b_gc4m, W_gc4s, b_gc4s,
                    W_tum[:D], W_tum[D:], b_tum, W_tls[:D], W_tls[D:], b_tls)

    user, ksum = _final(_shift(sm), _shift(sls), _shift(tm), _shift(tls))
    return user, ksum.reshape(())


# speedup vs baseline: 3.8250x; 1.2439x over previous
import functools
import math

import jax
import jax.numpy as jnp
from jax import lax
from jax.experimental import pallas as pl
from jax.experimental.pallas import tpu as pltpu
from jax.experimental.pallas import tpu_sc as plsc

N_USR = 29999
N_ITM = 20000
E = 480000
D = 128
ALPHA = 0.1
RATE = 0.5
TGT_USERS = 30000
TOTAL_USERS = 50000

BLK_CH = 8
E_PAD = 491520
U_PAD = 30208
I_PAD = 20224
HALF_U = U_PAD // 2
HALF_I = I_PAD // 2


def _leaky(x):
    return jnp.where(x > 0, x, ALPHA * x)


@functools.lru_cache(maxsize=None)
def _make_spmm(out_rows, half, ch):
    blk_e = ch * BLK_CH
    ept = E_PAD // 16
    n_blocks = ept // blk_e
    zpt = half // 16
    mesh = plsc.VectorSubcoreMesh(core_axis_name="c", subcore_axis_name="s")

    @functools.partial(
        pl.kernel,
        out_type=jax.ShapeDtypeStruct((out_rows, D), jnp.float32),
        mesh=mesh,
        scratch_types=[
            pltpu.VMEM((blk_e,), jnp.int32),
            pltpu.VMEM((blk_e,), jnp.int32),
            pltpu.VMEM((blk_e,), jnp.float32),
            pltpu.VMEM((ch, D), jnp.float32),
            pltpu.VMEM((ch, D), jnp.float32),
            pltpu.VMEM((ch,), jnp.int32),
            pltpu.VMEM((ch,), jnp.int32),
            pltpu.VMEM((ch,), jnp.float32),
            pltpu.VMEM((ch,), jnp.float32),
            pltpu.VMEM_SHARED((half, D), jnp.float32),
            pltpu.SemaphoreType.DMA,
            pltpu.SemaphoreType.DMA,
            pltpu.SemaphoreType.DMA,
            pltpu.SemaphoreType.DMA,
        ],
    )
    def spmm(x_hbm, cols_hbm, rows_hbm, vals_hbm, zeros_hbm, out_hbm,
             colsb, rowsb, valsb, g0, g1, l0, l1, v0, v1, acc,
             gs0, gs1, ss0, ss1):
        c = lax.axis_index("c")
        s = lax.axis_index("s")
        base = c * half
        gath = (g0, g1)
        locs = (l0, l1)
        vms = (v0, v1)
        gsem = (gs0, gs1)
        ssem = (ss0, ss1)
        pltpu.sync_copy(zeros_hbm.at[pl.ds(s * zpt, zpt)],
                        acc.at[pl.ds(s * zpt, zpt)])
        plsc.subcore_barrier()

        def block(b, carry):
            boff = s * ept + b * blk_e
            pltpu.sync_copy(cols_hbm.at[pl.ds(boff, blk_e)], colsb)
            pltpu.sync_copy(rows_hbm.at[pl.ds(boff, blk_e)], rowsb)
            pltpu.sync_copy(vals_hbm.at[pl.ds(boff, blk_e)], valsb)
            gd = {0: pltpu.async_copy(x_hbm.at[colsb.at[pl.ds(0, ch)]],
                                      gath[0], gsem[0])}
            sd = [None, None]
            for jj in range(BLK_CH):
                buf = jj & 1
                if jj < BLK_CH - 1:
                    nbuf = buf ^ 1
                    if jj >= 1:
                        sd[nbuf].wait()
                    gd[jj + 1] = pltpu.async_copy(
                        x_hbm.at[colsb.at[pl.ds((jj + 1) * ch, ch)]],
                        gath[nbuf], gsem[nbuf])
                for g in range(ch // 16):
                    slb = pl.ds(jj * ch + g * 16, 16)
                    sl = pl.ds(g * 16, 16)
                    lo = rowsb[slb] - base
                    inr = (lo >= 0) & (lo < half)
                    locs[buf][sl] = jnp.where(inr, lo, colsb[slb] & 8191)
                    vms[buf][sl] = jnp.where(inr, valsb[slb], 0.0)
                gd[jj].wait()

                def scale(g2, carry2, _vm=vms[buf], _gt=gath[buf]):
                    vv = _vm[pl.ds(g2 * 16, 16)]
                    for l in range(16):
                        v = vv[l]
                        e = g2 * 16 + l
                        for kk in range(D // 16):
                            sl2 = pl.ds(kk * 16, 16)
                            _gt[e, sl2] = _gt[e, sl2] * v
                    return carry2

                lax.fori_loop(0, ch // 16, scale, 0)
                sd[buf] = pltpu.async_copy(gath[buf], acc.at[locs[buf]],
                                           ssem[buf], add=True)
            sd[0].wait()
            sd[1].wait()
            return carry

        lax.fori_loop(0, n_blocks, block, 0)
        plsc.subcore_barrier()
        pltpu.sync_copy(acc.at[pl.ds(s * zpt, zpt)],
                        out_hbm.at[pl.ds(base + s * zpt, zpt)])

    return spmm


def _dense1(x, W, b):
    n = x.shape[0]
    br = 2528

    def body(x_ref, w_ref, b_ref, o_ref):
        acc = jnp.dot(x_ref[...], w_ref[...],
                      preferred_element_type=jnp.float32) + b_ref[...]
        o_ref[...] = _leaky(acc)

    return pl.pallas_call(
        body,
        grid=(n // br,),
        in_specs=[pl.BlockSpec((br, D), lambda i: (i, 0)),
                  pl.BlockSpec((D, D), lambda i: (0, 0)),
                  pl.BlockSpec((1, D), lambda i: (0, 0))],
        out_specs=pl.BlockSpec((br, D), lambda i: (i, 0)),
        out_shape=jax.ShapeDtypeStruct((n, D), jnp.float32),
    )(x, W, b.reshape(1, D))


def _head(S2, uf, Wm, bm, Ws, bs, Wc1a, Wc1b, bc1, Wc2a, Wc2b, bc2):
    n = S2.shape[0]
    br = 1888

    def body(s_ref, u_ref, wm, bm_, ws, bs_, wa1, wb1, bb1, wa2, wb2, bb2,
             o1, o2):
        sv = s_ref[...]
        uv = u_ref[...]
        hm = _leaky(jnp.dot(sv, wm[...], preferred_element_type=jnp.float32)
                    + bm_[...])
        hs = _leaky(jnp.dot(sv, ws[...], preferred_element_type=jnp.float32)
                    + bs_[...])
        ub1 = jnp.dot(uv, wb1[...], preferred_element_type=jnp.float32)
        ub2 = jnp.dot(uv, wb2[...], preferred_element_type=jnp.float32)
        o1[...] = jnp.dot(hm, wa1[...], preferred_element_type=jnp.float32) \
            + ub1 + bb1[...]
        o2[...] = jnp.dot(hs, wa2[...], preferred_element_type=jnp.float32) \
            + ub2 + bb2[...]

    mat = pl.BlockSpec((D, D), lambda i: (0, 0))
    vec = pl.BlockSpec((1, D), lambda i: (0, 0))
    blk = pl.BlockSpec((br, D), lambda i: (i, 0))
    return pl.pallas_call(
        body,
        grid=(n // br,),
        in_specs=[blk, blk, mat, vec, mat, vec, mat, mat, vec, mat, mat, vec],
        out_specs=[blk, blk],
        out_shape=[jax.ShapeDtypeStruct((n, D), jnp.float32),
                   jax.ShapeDtypeStruct((n, D), jnp.float32)],
    )(S2, uf, Wm, bm.reshape(1, D), Ws, bs.reshape(1, D),
      Wc1a, Wc1b, bc1.reshape(1, D), Wc2a, Wc2b, bc2.reshape(1, D))


def _final(smp, slsp, tmp, tlsp):
    br = 2000
    nb = TOTAL_USERS // br
    a2 = 0.1 + 0.9 * math.log(2.0)
    inv2s2 = 0.5 * math.exp(-2.0 * a2)

    def body(sm_ref, tm_ref, sl_ref, tl_ref, mean_ref, ks_ref):
        i = pl.program_id(0)
        both = i < 5
        t_only = (i >= 5) & (i < 15)
        sm = sm_ref[...]
        tm = tm_ref[...]
        sl = sl_ref[...]
        tl = tl_ref[...]
        mean = jnp.where(both, RATE * sm + (1 - RATE) * tm,
                         jnp.where(t_only, (1 - RATE) * tm, RATE * sm))
        ls = jnp.where(both, RATE * sl + (1 - RATE) * tl,
                       jnp.where(t_only, (1 - RATE) * tl, RATE * sl))
        mean_ref[...] = mean
        a1 = 0.1 + 0.9 * (jnp.maximum(ls, 0.0)
                          + jnp.log(1.0 + jnp.exp(-jnp.abs(ls))))
        kl = (a2 - a1) + (jnp.exp(2.0 * a1) + mean * mean) * inv2s2 - 0.5
        part = jnp.sum(kl)

        @pl.when(i == 0)
        def _():
            ks_ref[...] = jnp.zeros_like(ks_ref)

        ks_ref[...] = ks_ref[...] + part

        @pl.when(i == nb - 1)
        def _():
            ks_ref[...] = ks_ref[...] / float(TOTAL_USERS)

    def s_map(i):
        return (jnp.where(i < 5, i, jnp.where(i >= 15, i - 10, 0)), 0)

    def t_map(i):
        return (jnp.where(i < 15, i, 0), 0)

    blk_s = pl.BlockSpec((br, D), s_map)
    blk_t = pl.BlockSpec((br, D), t_map)
    return pl.pallas_call(
        body,
        grid=(nb,),
        in_specs=[blk_s, blk_t, blk_s, blk_t],
        out_specs=[pl.BlockSpec((br, D), lambda i: (i, 0)),
                   pl.BlockSpec((1, 1), lambda i: (0, 0))],
        out_shape=[jax.ShapeDtypeStruct((TOTAL_USERS, D), jnp.float32),
                   jax.ShapeDtypeStruct((1, 1), jnp.float32)],
    )(smp, tmp, slsp, tlsp)


def _pad_rows(x, n):
    return jnp.pad(x, ((0, n - x.shape[0]), (0, 0)))


def _pad_edges(idx, n_valid):
    pad = E_PAD - E
    filler = (jnp.arange(pad, dtype=jnp.int32) % min(997, n_valid))
    return jnp.concatenate([idx.astype(jnp.int32), filler])


def _shift(x):
    return jnp.concatenate([jnp.zeros((1, D), jnp.float32), x[:TGT_USERS - 1]])


def kernel(source_ufea, target_ufea, source_user_idx, source_item_idx,
           source_vals, target_user_idx, target_item_idx, target_vals,
           W_gc1, b_gc1, W_gc2, b_gc2, W_gc3m, b_gc3m, W_gc3s, b_gc3s,
           W_gc4m, b_gc4m, W_gc4s, b_gc4s, W_sum, b_sum, W_sls, b_sls,
           W_tum, b_tum, W_tls, b_tls):
    su = _pad_rows(source_ufea, U_PAD)
    tu = _pad_rows(target_ufea, U_PAD)
    s_ui = _pad_edges(source_user_idx, N_USR)
    s_ii = _pad_edges(source_item_idx, N_ITM)
    t_ui = _pad_edges(target_user_idx, N_USR)
    t_ii = _pad_edges(target_item_idx, N_ITM)
    vpad = jnp.zeros((E_PAD - E,), jnp.float32)
    s_v = jnp.concatenate([source_vals, vpad])
    t_v = jnp.concatenate([target_vals, vpad])
    zeros_i = jnp.zeros((HALF_I, D), jnp.float32)
    zeros_u = jnp.zeros((HALF_U, D), jnp.float32)
    _spmm_item = _make_spmm(I_PAD, HALF_I, 128)
    _spmm_user = _make_spmm(U_PAD, HALF_U, 32)

    S1s = _spmm_item(su, s_ui, s_ii, s_v, zeros_i)
    S1t = _spmm_item(tu, t_ui, t_ii, t_v, zeros_i)
    s_ho = _dense1(S1s, W_gc1, b_gc1)
    t_ho = _dense1(S1t, W_gc2, b_gc2)
    S2s = _spmm_user(s_ho, s_ii, s_ui, s_v, zeros_u)
    S2t = _spmm_user(t_ho, t_ii, t_ui, t_v, zeros_u)

    sm, sls = _head(S2s, su, W_gc3m, b_gc3m, W_gc3s, b_gc3s,
                    W_sum[:D], W_sum[D:], b_sum, W_sls[:D], W_sls[D:], b_sls)
    tm, tls = _head(S2t, tu, W_gc4m, b_gc4m, W_gc4s, b_gc4s,
                    W_tum[:D], W_tum[D:], b_tum, W_tls[:D], W_tls[D:], b_tls)

    user, ksum = _final(_shift(sm), _shift(sls), _shift(tm), _shift(tls))
    return user, ksum.reshape(())


# speedup vs baseline: 3.8304x; 1.0014x over previous
import functools
import math

import jax
import jax.numpy as jnp
from jax import lax
from jax.experimental import pallas as pl
from jax.experimental.pallas import tpu as pltpu
from jax.experimental.pallas import tpu_sc as plsc

N_USR = 29999
N_ITM = 20000
E = 480000
D = 128
ALPHA = 0.1
RATE = 0.5
TGT_USERS = 30000
TOTAL_USERS = 50000

BLK_CH = 8
E_PAD = 491520
U_PAD = 30208
I_PAD = 20224
HALF_U = U_PAD // 2
HALF_I = I_PAD // 2


def _leaky(x):
    return jnp.where(x > 0, x, ALPHA * x)


@functools.lru_cache(maxsize=None)
def _make_spmm(out_rows, half, ch, shift=0):
    blk_e = ch * BLK_CH
    ept = E_PAD // 16
    n_blocks = ept // blk_e
    zpt = half // 16
    mesh = plsc.VectorSubcoreMesh(core_axis_name="c", subcore_axis_name="s")

    @functools.partial(
        pl.kernel,
        out_type=jax.ShapeDtypeStruct((out_rows, D), jnp.float32),
        mesh=mesh,
        scratch_types=[
            pltpu.VMEM((blk_e,), jnp.int32),
            pltpu.VMEM((blk_e,), jnp.int32),
            pltpu.VMEM((blk_e,), jnp.float32),
            pltpu.VMEM((ch, D), jnp.float32),
            pltpu.VMEM((ch, D), jnp.float32),
            pltpu.VMEM((ch,), jnp.int32),
            pltpu.VMEM((ch,), jnp.int32),
            pltpu.VMEM((ch,), jnp.float32),
            pltpu.VMEM((ch,), jnp.float32),
            pltpu.VMEM_SHARED((half, D), jnp.float32),
            pltpu.SemaphoreType.DMA,
            pltpu.SemaphoreType.DMA,
            pltpu.SemaphoreType.DMA,
            pltpu.SemaphoreType.DMA,
        ],
    )
    def spmm(x_hbm, cols_hbm, rows_hbm, vals_hbm, zeros_hbm, out_hbm,
             colsb, rowsb, valsb, g0, g1, l0, l1, v0, v1, acc,
             gs0, gs1, ss0, ss1):
        c = lax.axis_index("c")
        s = lax.axis_index("s")
        base = c * half
        gath = (g0, g1)
        locs = (l0, l1)
        vms = (v0, v1)
        gsem = (gs0, gs1)
        ssem = (ss0, ss1)
        pltpu.sync_copy(zeros_hbm.at[pl.ds(s * zpt, zpt)],
                        acc.at[pl.ds(s * zpt, zpt)])
        plsc.subcore_barrier()

        def block(b, carry):
            boff = s * ept + b * blk_e
            pltpu.sync_copy(cols_hbm.at[pl.ds(boff, blk_e)], colsb)
            pltpu.sync_copy(rows_hbm.at[pl.ds(boff, blk_e)], rowsb)
            pltpu.sync_copy(vals_hbm.at[pl.ds(boff, blk_e)], valsb)
            gd = {0: pltpu.async_copy(x_hbm.at[colsb.at[pl.ds(0, ch)]],
                                      gath[0], gsem[0])}
            sd = [None, None]
            for jj in range(BLK_CH):
                buf = jj & 1
                if jj < BLK_CH - 1:
                    nbuf = buf ^ 1
                    if jj >= 1:
                        sd[nbuf].wait()
                    gd[jj + 1] = pltpu.async_copy(
                        x_hbm.at[colsb.at[pl.ds((jj + 1) * ch, ch)]],
                        gath[nbuf], gsem[nbuf])
                for g in range(ch // 16):
                    slb = pl.ds(jj * ch + g * 16, 16)
                    sl = pl.ds(g * 16, 16)
                    lo = rowsb[slb] + shift - base
                    inr = (lo >= 0) & (lo < half)
                    locs[buf][sl] = jnp.where(inr, lo, colsb[slb] & 8191)
                    vms[buf][sl] = jnp.where(inr, valsb[slb], 0.0)
                gd[jj].wait()

                def scale(g2, carry2, _vm=vms[buf], _gt=gath[buf]):
                    vv = _vm[pl.ds(g2 * 16, 16)]
                    for l in range(16):
                        v = vv[l]
                        e = g2 * 16 + l
                        for kk in range(D // 16):
                            sl2 = pl.ds(kk * 16, 16)
                            _gt[e, sl2] = _gt[e, sl2] * v
                    return carry2

                lax.fori_loop(0, ch // 16, scale, 0)
                sd[buf] = pltpu.async_copy(gath[buf], acc.at[locs[buf]],
                                           ssem[buf], add=True)
            sd[0].wait()
            sd[1].wait()
            return carry

        lax.fori_loop(0, n_blocks, block, 0)
        plsc.subcore_barrier()
        pltpu.sync_copy(acc.at[pl.ds(s * zpt, zpt)],
                        out_hbm.at[pl.ds(base + s * zpt, zpt)])

    return spmm


def _dense1(x, W, b):
    n = x.shape[0]
    br = 2528

    def body(x_ref, w_ref, b_ref, o_ref):
        acc = jnp.dot(x_ref[...], w_ref[...],
                      preferred_element_type=jnp.float32) + b_ref[...]
        o_ref[...] = _leaky(acc)

    return pl.pallas_call(
        body,
        grid=(n // br,),
        in_specs=[pl.BlockSpec((br, D), lambda i: (i, 0)),
                  pl.BlockSpec((D, D), lambda i: (0, 0)),
                  pl.BlockSpec((1, D), lambda i: (0, 0))],
        out_specs=pl.BlockSpec((br, D), lambda i: (i, 0)),
        out_shape=jax.ShapeDtypeStruct((n, D), jnp.float32),
    )(x, W, b.reshape(1, D))


def _head(S2, uf, Wm, bm, Ws, bs, Wc1a, Wc1b, bc1, Wc2a, Wc2b, bc2):
    n = S2.shape[0]
    br = 1888

    def body(s_ref, u_ref, wm, bm_, ws, bs_, wa1, wb1, bb1, wa2, wb2, bb2,
             o1, o2):
        sv = s_ref[...]
        uv = u_ref[...]
        hm = _leaky(jnp.dot(sv, wm[...], preferred_element_type=jnp.float32)
                    + bm_[...])
        hs = _leaky(jnp.dot(sv, ws[...], preferred_element_type=jnp.float32)
                    + bs_[...])
        ub1 = jnp.dot(uv, wb1[...], preferred_element_type=jnp.float32)
        ub2 = jnp.dot(uv, wb2[...], preferred_element_type=jnp.float32)
        o1[...] = jnp.dot(hm, wa1[...], preferred_element_type=jnp.float32) \
            + ub1 + bb1[...]
        o2[...] = jnp.dot(hs, wa2[...], preferred_element_type=jnp.float32) \
            + ub2 + bb2[...]

    mat = pl.BlockSpec((D, D), lambda i: (0, 0))
    vec = pl.BlockSpec((1, D), lambda i: (0, 0))
    blk = pl.BlockSpec((br, D), lambda i: (i, 0))
    return pl.pallas_call(
        body,
        grid=(n // br,),
        in_specs=[blk, blk, mat, vec, mat, vec, mat, mat, vec, mat, mat, vec],
        out_specs=[blk, blk],
        out_shape=[jax.ShapeDtypeStruct((n, D), jnp.float32),
                   jax.ShapeDtypeStruct((n, D), jnp.float32)],
    )(S2, uf, Wm, bm.reshape(1, D), Ws, bs.reshape(1, D),
      Wc1a, Wc1b, bc1.reshape(1, D), Wc2a, Wc2b, bc2.reshape(1, D))


def _final(smp, slsp, tmp, tlsp):
    br = 2000
    nb = TOTAL_USERS // br
    a2 = 0.1 + 0.9 * math.log(2.0)
    inv2s2 = 0.5 * math.exp(-2.0 * a2)

    def body(sm_ref, tm_ref, sl_ref, tl_ref, mean_ref, ks_ref):
        i = pl.program_id(0)
        both = i < 5
        t_only = (i >= 5) & (i < 15)
        sm = sm_ref[...]
        tm = tm_ref[...]
        sl = sl_ref[...]
        tl = tl_ref[...]
        mean = jnp.where(both, RATE * sm + (1 - RATE) * tm,
                         jnp.where(t_only, (1 - RATE) * tm, RATE * sm))
        ls = jnp.where(both, RATE * sl + (1 - RATE) * tl,
                       jnp.where(t_only, (1 - RATE) * tl, RATE * sl))
        row0 = (i == 0) & (jax.lax.broadcasted_iota(jnp.int32, mean.shape, 0)
                           == 0)
        mean = jnp.where(row0, 0.0, mean)
        ls = jnp.where(row0, 0.0, ls)
        mean_ref[...] = mean
        a1 = 0.1 + 0.9 * (jnp.maximum(ls, 0.0)
                          + jnp.log(1.0 + jnp.exp(-jnp.abs(ls))))
        kl = (a2 - a1) + (jnp.exp(2.0 * a1) + mean * mean) * inv2s2 - 0.5
        part = jnp.sum(kl)

        @pl.when(i == 0)
        def _():
            ks_ref[...] = jnp.zeros_like(ks_ref)

        ks_ref[...] = ks_ref[...] + part

        @pl.when(i == nb - 1)
        def _():
            ks_ref[...] = ks_ref[...] / float(TOTAL_USERS)

    def s_map(i):
        return (jnp.where(i < 5, i, jnp.where(i >= 15, i - 10, 0)), 0)

    def t_map(i):
        return (jnp.where(i < 15, i, 0), 0)

    blk_s = pl.BlockSpec((br, D), s_map)
    blk_t = pl.BlockSpec((br, D), t_map)
    return pl.pallas_call(
        body,
        grid=(nb,),
        in_specs=[blk_s, blk_t, blk_s, blk_t],
        out_specs=[pl.BlockSpec((br, D), lambda i: (i, 0)),
                   pl.BlockSpec((1, 1), lambda i: (0, 0))],
        out_shape=[jax.ShapeDtypeStruct((TOTAL_USERS, D), jnp.float32),
                   jax.ShapeDtypeStruct((1, 1), jnp.float32)],
    )(smp, tmp, slsp, tlsp)


def _pad_rows(x, n):
    return jnp.pad(x, ((0, n - x.shape[0]), (0, 0)))


def _pad_edges(idx, n_valid):
    pad = E_PAD - E
    filler = (jnp.arange(pad, dtype=jnp.int32) % min(997, n_valid))
    return jnp.concatenate([idx.astype(jnp.int32), filler])


def _shift(x):
    return jnp.concatenate([jnp.zeros((1, D), jnp.float32), x[:TGT_USERS - 1]])


def kernel(source_ufea, target_ufea, source_user_idx, source_item_idx,
           source_vals, target_user_idx, target_item_idx, target_vals,
           W_gc1, b_gc1, W_gc2, b_gc2, W_gc3m, b_gc3m, W_gc3s, b_gc3s,
           W_gc4m, b_gc4m, W_gc4s, b_gc4s, W_sum, b_sum, W_sls, b_sls,
           W_tum, b_tum, W_tls, b_tls):
    su = _pad_rows(source_ufea, U_PAD)
    tu = _pad_rows(target_ufea, U_PAD)
    z1 = jnp.zeros((1, D), jnp.float32)
    su_sh = _pad_rows(jnp.concatenate([z1, source_ufea]), U_PAD)
    tu_sh = _pad_rows(jnp.concatenate([z1, target_ufea]), U_PAD)
    s_ui = _pad_edges(source_user_idx, N_USR)
    s_ii = _pad_edges(source_item_idx, N_ITM)
    t_ui = _pad_edges(target_user_idx, N_USR)
    t_ii = _pad_edges(target_item_idx, N_ITM)
    vpad = jnp.zeros((E_PAD - E,), jnp.float32)
    s_v = jnp.concatenate([source_vals, vpad])
    t_v = jnp.concatenate([target_vals, vpad])
    zeros_i = jnp.zeros((HALF_I, D), jnp.float32)
    zeros_u = jnp.zeros((HALF_U, D), jnp.float32)
    _spmm_item = _make_spmm(I_PAD, HALF_I, 128)
    _spmm_user = _make_spmm(U_PAD, HALF_U, 32, 1)

    S1s = _spmm_item(su, s_ui, s_ii, s_v, zeros_i)
    S1t = _spmm_item(tu, t_ui, t_ii, t_v, zeros_i)
    s_ho = _dense1(S1s, W_gc1, b_gc1)
    t_ho = _dense1(S1t, W_gc2, b_gc2)
    S2s = _spmm_user(s_ho, s_ii, s_ui, s_v, zeros_u)
    S2t = _spmm_user(t_ho, t_ii, t_ui, t_v, zeros_u)

    sm, sls = _head(S2s, su_sh, W_gc3m, b_gc3m, W_gc3s, b_gc3s,
                    W_sum[:D], W_sum[D:], b_sum, W_sls[:D], W_sls[D:], b_sls)
    tm, tls = _head(S2t, tu_sh, W_gc4m, b_gc4m, W_gc4s, b_gc4s,
                    W_tum[:D], W_tum[D:], b_tum, W_tls[:D], W_tls[D:], b_tls)

    user, ksum = _final(sm, sls, tm, tls)
    return user, ksum.reshape(())
